# baseline (speedup 1.0000x reference)
import functools

import jax
import jax.numpy as jnp
from jax import lax
from jax.experimental import pallas as pl
from jax.experimental.pallas import tpu as pltpu
from jax.experimental.pallas import tpu_sc as plsc

F32 = jnp.float32
I32 = jnp.int32
LANE = 16
B = 64
NW = 32


def _k1_body(x_ref, wt_ref, a1_ref, a2_ref, h_ref, s1_ref, s2_ref):
    h = jnp.dot(x_ref[...], wt_ref[...], preferred_element_type=F32)
    h_ref[...] = h
    s1_ref[...] = jnp.dot(h, a1_ref[...], preferred_element_type=F32)
    s2_ref[...] = jnp.dot(h, a2_ref[...], preferred_element_type=F32)


def _k3_body(op_ref, dp_ref, r_ref, x_ref, g_ref, b_ref, o_ref):
    den = dp_ref[0] + dp_ref[1] + 1e-16
    den_e = jnp.dot(den, r_ref[...], preferred_element_type=F32)
    o = (op_ref[0] + op_ref[1]) / den_e
    mu = jnp.mean(o, axis=-1, keepdims=True)
    d = o - mu
    var = jnp.mean(d * d, axis=-1, keepdims=True)
    y = d * lax.rsqrt(var + 1e-5) * g_ref[...] + b_ref[...]
    y = jnp.where(y > 0.0, y, jnp.exp(jnp.minimum(y, 0.0)) - 1.0)
    o_ref[...] = y + x_ref[...]


def kernel(x, edge_index, W, att, ln_gamma, ln_beta):
    N, C = x.shape
    H = att.shape[1]
    D = att.shape[2] // 2
    HD = H * D
    E = edge_index.shape[1]

    CH = 6 * (-(-E // (6 * B * NW)))
    EP = B * NW * CH
    NP = -(-(N + 1) // 2048) * 2048
    ZR = NP // 16

    ei = edge_index.astype(I32)
    src = ei[0]
    dst = ei[1]
    dst3 = jnp.concatenate([dst, jnp.full((EP - E,), N, I32)]).reshape(NW, CH, B)
    src3 = jnp.concatenate([src, jnp.zeros((EP - E,), I32)]).reshape(NW, CH, B)
    e4 = jnp.stack([dst3, src3], axis=2).reshape(NW * CH, 2, B)
    xP = jnp.concatenate([x, jnp.zeros((NP - N, C), F32)])

    attd = att[0, :, :D]
    atts = att[0, :, D:]
    eye = jnp.eye(H, dtype=F32)
    Ad = (eye[:, None, :] * attd[:, :, None]).reshape(HD, H)
    As = (eye[:, None, :] * atts[:, :, None]).reshape(HD, H)
    A1 = jnp.concatenate([Ad, As], axis=1)
    A2 = jnp.concatenate([As, Ad], axis=1)
    R = (eye[:, None, :] * jnp.ones((H, D, H), F32)).reshape(HD, H).T
    R = jnp.concatenate([R, jnp.zeros((LANE - H, HD), F32)], axis=0)

    hP, s1, s2 = pl.pallas_call(
        _k1_body,
        out_shape=(
            jax.ShapeDtypeStruct((NP, HD), F32),
            jax.ShapeDtypeStruct((NP, LANE), F32),
            jax.ShapeDtypeStruct((NP, LANE), F32),
        ),
    )(xP, W.T, A1, A2)

    mesh = plsc.VectorSubcoreMesh(core_axis_name="c", subcore_axis_name="s")
    sc_params = pltpu.CompilerParams(use_tc_tiling_on_sc=False)

    def _wait(dummy_src, buf, sem):
        pltpu.make_async_copy(dummy_src, buf, sem).wait()

    @functools.partial(
        pl.kernel,
        mesh=mesh,
        out_type=(
            jax.ShapeDtypeStruct((2, NP, LANE), F32),
            jax.ShapeDtypeStruct((2, NP, HD), F32),
        ),
        scratch_types=[
            pltpu.VMEM((6, 2, B), I32),
            pltpu.VMEM((B, LANE), F32),
            pltpu.VMEM((B, LANE), F32),
            pltpu.VMEM((B, LANE), F32),
            pltpu.VMEM((B, LANE), F32),
            pltpu.VMEM((B, LANE), F32),
            pltpu.VMEM((B, LANE), F32),
            pltpu.VMEM((B, HD), F32),
            pltpu.VMEM((B, HD), F32),
            pltpu.VMEM((B, HD), F32),
            pltpu.VMEM_SHARED((NP, LANE), F32),
            pltpu.VMEM_SHARED((NP, HD), F32),
        ] + [pltpu.SemaphoreType.DMA] * 9,
        compiler_params=sc_params,
    )
    def k2(s1_hbm, s2_hbm, h_hbm, e4_hbm, dp_out, op_out,
           idq, g1a, g1b, g1c, g2a, g2b, g2c, hva, hvb, hvc, dsh, osh,
           sema, semb, semc, isem0, isem1, isem2, isem3, isem4, isem5):
        cid = lax.axis_index("c")
        sid = lax.axis_index("s")
        wid = cid * 16 + sid
        g1 = (g1a, g1b, g1c)
        g2 = (g2a, g2b, g2c)
        hv = (hva, hvb, hvc)
        sem = (sema, semb, semc)
        isem = (isem0, isem1, isem2, isem3, isem4, isem5)

        @pl.loop(0, B)
        def _(i):
            for cc in range(HD // LANE):
                hva[i, pl.ds(cc * LANE, LANE)] = jnp.zeros((LANE,), F32)
            g1a[i, :] = jnp.zeros((LANE,), F32)
        for k in range(ZR // B):
            pltpu.sync_copy(hva, osh.at[pl.ds(sid * ZR + k * B, B)])
            pltpu.sync_copy(g1a, dsh.at[pl.ds(sid * ZR + k * B, B)])
        ZREM = ZR - (ZR // B) * B
        if ZREM:
            rows = pl.ds(sid * ZR + (ZR // B) * B, ZREM)
            pltpu.sync_copy(hva.at[pl.ds(0, ZREM)], osh.at[rows])
            pltpu.sync_copy(g1a.at[pl.ds(0, ZREM)], dsh.at[rows])
        plsc.subcore_barrier()

        def idxload(j, s):
            @pl.when(j < CH)
            def _():
                pltpu.async_copy(e4_hbm.at[wid * CH + j], idq.at[s], isem[s])

        def gathers(j, p, s, wait_idx=True):
            @pl.when(j < CH)
            def _():
                if wait_idx:
                    _wait(e4_hbm.at[0], idq.at[s], isem[s])
                pltpu.async_copy(s1_hbm.at[idq.at[s, 0]], g1[p], sem[p])
                pltpu.async_copy(s2_hbm.at[idq.at[s, 1]], g2[p], sem[p])
                pltpu.async_copy(h_hbm.at[idq.at[s, 1]], hv[p], sem[p])

        def compute(j, p, s):
            _wait(s1_hbm.at[pl.ds(0, B)], g1[p], sem[p])
            _wait(s2_hbm.at[pl.ds(0, B)], g2[p], sem[p])
            _wait(h_hbm.at[pl.ds(0, B)], hv[p], sem[p])

            @pl.loop(0, B)
            def _(i):
                v = g1[p][i, :] + g2[p][i, :]
                v = jnp.where(v >= 0.0, v, 0.2 * v)
                v = jnp.exp(v)
                g1[p][i, :] = v
                for hd in range(H):
                    sl = pl.ds(hd * LANE, LANE)
                    hv[p][i, sl] = hv[p][i, sl] * v[hd]

            pltpu.sync_copy(g1[p], dsh.at[idq.at[s, 0]], add=True)
            pltpu.sync_copy(hv[p], osh.at[idq.at[s, 0]], add=True)

        for t in range(3):
            pltpu.sync_copy(e4_hbm.at[wid * CH + t], idq.at[t])
        for t in range(3):
            gathers(t, t, t, wait_idx=False)
        for t in range(3, 6):
            idxload(t, t)

        @pl.loop(0, CH, step=6)
        def _(j):
            for t in range(6):
                compute(j + t, t % 3, t)
                idxload(j + t + 6, t)
                gathers(j + t + 3, t % 3, (t + 3) % 6)

        plsc.subcore_barrier()
        for k in range(ZR // 128):
            rows = pl.ds(sid * ZR + k * 128, 128)
            pltpu.sync_copy(dsh.at[rows], dp_out.at[cid, rows])
            pltpu.sync_copy(osh.at[rows], op_out.at[cid, rows])

    dpart, opart = k2(s1, s2, hP, e4)

    out = pl.pallas_call(
        _k3_body,
        out_shape=jax.ShapeDtypeStruct((NP, HD), F32),
    )(opart, dpart, R, xP, ln_gamma, ln_beta)

    return out[:N]


# speedup vs baseline: 1.2208x; 1.2208x over previous
import functools

import jax
import jax.numpy as jnp
from jax import lax
from jax.experimental import pallas as pl
from jax.experimental.pallas import tpu as pltpu
from jax.experimental.pallas import tpu_sc as plsc

F32 = jnp.float32
I32 = jnp.int32
LANE = 16
B = 64
NW = 32


def _k1_body(x_ref, wt_ref, a1_ref, a2_ref, h_ref, s1_ref, s2_ref):
    h = jnp.dot(x_ref[...], wt_ref[...], preferred_element_type=F32)
    h_ref[...] = h
    s1_ref[...] = jnp.dot(h, a1_ref[...], preferred_element_type=F32)
    s2_ref[...] = jnp.dot(h, a2_ref[...], preferred_element_type=F32)


def _k3_body(op_ref, dp_ref, r_ref, x_ref, g_ref, b_ref, o_ref):
    den = dp_ref[0] + dp_ref[1] + 1e-16
    den_e = jnp.dot(den, r_ref[...], preferred_element_type=F32)
    o = (op_ref[0] + op_ref[1]) / den_e
    mu = jnp.mean(o, axis=-1, keepdims=True)
    d = o - mu
    var = jnp.mean(d * d, axis=-1, keepdims=True)
    y = d * lax.rsqrt(var + 1e-5) * g_ref[...] + b_ref[...]
    y = jnp.where(y > 0.0, y, jnp.exp(jnp.minimum(y, 0.0)) - 1.0)
    o_ref[...] = y + x_ref[...]


def kernel(x, edge_index, W, att, ln_gamma, ln_beta):
    N, C = x.shape
    H = att.shape[1]
    D = att.shape[2] // 2
    HD = H * D
    E = edge_index.shape[1]

    CH = 4 * (-(-E // (4 * B * NW)))
    EP = B * NW * CH
    NP = -(-(N + 1) // 2048) * 2048
    ZR = NP // 16

    ei = edge_index.astype(I32)
    src = ei[0]
    dst = ei[1]
    dst3 = jnp.concatenate([dst, jnp.full((EP - E,), N, I32)]).reshape(NW, CH, B)
    src3 = jnp.concatenate([src, jnp.zeros((EP - E,), I32)]).reshape(NW, CH, B)
    e4 = jnp.stack([dst3, src3], axis=2).reshape(NW * CH, 2, B)
    xP = jnp.concatenate([x, jnp.zeros((NP - N, C), F32)])

    attd = att[0, :, :D]
    atts = att[0, :, D:]
    eye = jnp.eye(H, dtype=F32)
    Ad = (eye[:, None, :] * attd[:, :, None]).reshape(HD, H)
    As = (eye[:, None, :] * atts[:, :, None]).reshape(HD, H)
    A1 = jnp.concatenate([Ad, As], axis=1)
    A2 = jnp.concatenate([As, Ad], axis=1)
    R = (eye[:, None, :] * jnp.ones((H, D, H), F32)).reshape(HD, H).T
    R = jnp.concatenate([R, jnp.zeros((LANE - H, HD), F32)], axis=0)

    hP, s1, s2 = pl.pallas_call(
        _k1_body,
        out_shape=(
            jax.ShapeDtypeStruct((NP, HD), F32),
            jax.ShapeDtypeStruct((NP, LANE), F32),
            jax.ShapeDtypeStruct((NP, LANE), F32),
        ),
    )(xP, W.T, A1, A2)

    mesh = plsc.VectorSubcoreMesh(core_axis_name="c", subcore_axis_name="s")
    sc_params = pltpu.CompilerParams(use_tc_tiling_on_sc=False)

    def _wait(dummy_src, buf, sem):
        pltpu.make_async_copy(dummy_src, buf, sem).wait()

    @functools.partial(
        pl.kernel,
        mesh=mesh,
        out_type=(
            jax.ShapeDtypeStruct((2, NP, LANE), F32),
            jax.ShapeDtypeStruct((2, NP, HD), F32),
        ),
        scratch_types=[
            pltpu.VMEM((4, 2, B), I32),
            pltpu.VMEM((B, LANE), F32),
            pltpu.VMEM((B, LANE), F32),
            pltpu.VMEM((B, LANE), F32),
            pltpu.VMEM((B, LANE), F32),
            pltpu.VMEM((B, HD), F32),
            pltpu.VMEM((B, HD), F32),
            pltpu.VMEM_SHARED((NP, LANE), F32),
            pltpu.VMEM_SHARED((NP, HD), F32),
        ] + [pltpu.SemaphoreType.DMA] * 6,
        compiler_params=sc_params,
    )
    def k2(s1_hbm, s2_hbm, h_hbm, e4_hbm, dp_out, op_out,
           idq, g1a, g1b, g2a, g2b, hva, hvb, dsh, osh,
           sema, semb, isem0, isem1, isem2, isem3):
        cid = lax.axis_index("c")
        sid = lax.axis_index("s")
        wid = cid * 16 + sid
        g1 = (g1a, g1b)
        g2 = (g2a, g2b)
        hv = (hva, hvb)
        sem = (sema, semb)
        isem = (isem0, isem1, isem2, isem3)

        @pl.loop(0, B)
        def _(i):
            for cc in range(HD // LANE):
                hva[i, pl.ds(cc * LANE, LANE)] = jnp.zeros((LANE,), F32)
            g1a[i, :] = jnp.zeros((LANE,), F32)
        for k in range(ZR // B):
            pltpu.sync_copy(hva, osh.at[pl.ds(sid * ZR + k * B, B)])
            pltpu.sync_copy(g1a, dsh.at[pl.ds(sid * ZR + k * B, B)])
        ZREM = ZR - (ZR // B) * B
        if ZREM:
            rows = pl.ds(sid * ZR + (ZR // B) * B, ZREM)
            pltpu.sync_copy(hva.at[pl.ds(0, ZREM)], osh.at[rows])
            pltpu.sync_copy(g1a.at[pl.ds(0, ZREM)], dsh.at[rows])
        plsc.subcore_barrier()

        def idxload(j, s):
            @pl.when(j < CH)
            def _():
                pltpu.async_copy(e4_hbm.at[wid * CH + j], idq.at[s], isem[s])

        def gathers(j, p, s, wait_idx=True):
            @pl.when(j < CH)
            def _():
                if wait_idx:
                    _wait(e4_hbm.at[0], idq.at[s], isem[s])
                pltpu.async_copy(s1_hbm.at[idq.at[s, 0]], g1[p], sem[p])
                pltpu.async_copy(s2_hbm.at[idq.at[s, 1]], g2[p], sem[p])
                pltpu.async_copy(h_hbm.at[idq.at[s, 1]], hv[p], sem[p])

        def compute(j, p, s):
            _wait(s1_hbm.at[pl.ds(0, B)], g1[p], sem[p])
            _wait(s2_hbm.at[pl.ds(0, B)], g2[p], sem[p])
            _wait(h_hbm.at[pl.ds(0, B)], hv[p], sem[p])

            @pl.loop(0, B)
            def _(i):
                v = g1[p][i, :] + g2[p][i, :]
                v = jnp.where(v >= 0.0, v, 0.2 * v)
                v = jnp.exp(v)
                g1[p][i, :] = v
                for hd in range(H):
                    sl = pl.ds(hd * LANE, LANE)
                    hv[p][i, sl] = hv[p][i, sl] * v[hd]

            pltpu.sync_copy(g1[p], dsh.at[idq.at[s, 0]], add=True)
            pltpu.sync_copy(hv[p], osh.at[idq.at[s, 0]], add=True)

        pltpu.sync_copy(e4_hbm.at[wid * CH], idq.at[0])
        pltpu.sync_copy(e4_hbm.at[wid * CH + 1], idq.at[1])
        gathers(0, 0, 0, wait_idx=False)
        gathers(1, 1, 1, wait_idx=False)
        idxload(2, 2)
        idxload(3, 3)

        @pl.loop(0, CH, step=4)
        def _(j):
            compute(j, 0, 0)
            idxload(j + 4, 0)
            gathers(j + 2, 0, 2)
            compute(j + 1, 1, 1)
            idxload(j + 5, 1)
            gathers(j + 3, 1, 3)
            compute(j + 2, 0, 2)
            idxload(j + 6, 2)
            gathers(j + 4, 0, 0)
            compute(j + 3, 1, 3)
            idxload(j + 7, 3)
            gathers(j + 5, 1, 1)

        plsc.subcore_barrier()
        for k in range(ZR // 128):
            rows = pl.ds(sid * ZR + k * 128, 128)
            pltpu.sync_copy(dsh.at[rows], dp_out.at[cid, rows])
            pltpu.sync_copy(osh.at[rows], op_out.at[cid, rows])

    dpart, opart = k2(s1, s2, hP, e4)

    out = pl.pallas_call(
        _k3_body,
        out_shape=jax.ShapeDtypeStruct((NP, HD), F32),
    )(opart, dpart, R, xP, ln_gamma, ln_beta)

    return out[:N]
